# baseline (speedup 1.0000x reference)
import functools

import jax
import jax.numpy as jnp
from jax import lax
from jax.experimental import pallas as pl
from jax.experimental.pallas import tpu as pltpu
from jax.experimental.pallas import tpu_sc as plsc

_LANES = 16


@functools.lru_cache(maxsize=None)
def _build_bag(B, L, V, D):
    info = plsc.get_sparse_core_info()
    NW = info.num_cores * info.num_subcores
    assert B % NW == 0
    b_per_w = B // NW
    n_idx = b_per_w * L
    chunks = []
    off = 0
    while off < n_idx:
        c = min(128, n_idx - off)
        chunks.append((off, c))
        off += c
    n_col = D // _LANES

    mesh = plsc.VectorSubcoreMesh(core_axis_name="c", subcore_axis_name="s")

    @functools.partial(
        pl.kernel,
        mesh=mesh,
        out_type=jax.ShapeDtypeStruct((B, D), jnp.float32),
        scratch_types=[
            pltpu.VMEM((n_idx,), jnp.int32),
            pltpu.VMEM((n_idx, D), jnp.float32),
            pltpu.VMEM((b_per_w, D), jnp.float32),
            pltpu.SemaphoreType.DMA,
        ],
        compiler_params=pltpu.CompilerParams(use_tc_tiling_on_sc=False),
    )
    def bag(x_hbm, embed_hbm, out_hbm, idx_v, rows_v, acc_v, sem):
        wid = lax.axis_index("s") * info.num_cores + lax.axis_index("c")
        base = wid * b_per_w
        pltpu.sync_copy(x_hbm.at[pl.ds(base * L, n_idx)], idx_v)
        copies = [
            pltpu.async_copy(
                embed_hbm.at[idx_v.at[pl.ds(o, c)]],
                rows_v.at[pl.ds(o, c)],
                sem,
            )
            for o, c in chunks
        ]
        for cp in copies:
            cp.wait()

        def body_i(i, _):
            r0 = i * L

            def body_j(j, acc):
                r = r0 + j
                return tuple(
                    acc[k] + rows_v[r, pl.ds(_LANES * k, _LANES)]
                    for k in range(n_col)
                )

            acc = lax.fori_loop(
                0, L, body_j,
                tuple(jnp.zeros((_LANES,), jnp.float32) for _ in range(n_col)),
            )
            for k in range(n_col):
                acc_v[i, pl.ds(_LANES * k, _LANES)] = acc[k]
            return 0

        lax.fori_loop(0, b_per_w, body_i, 0)
        pltpu.sync_copy(acc_v, out_hbm.at[pl.ds(base, b_per_w)])

    return bag


def _mm_body(s_ref, w_ref, b_ref, o_ref):
    o_ref[...] = lax.dot_general(
        s_ref[...], w_ref[...],
        dimension_numbers=(((1,), (1,)), ((), ())),
        preferred_element_type=jnp.float32,
    ) + b_ref[...]


@functools.lru_cache(maxsize=None)
def _build_mm(B, V, D, vt=2048):
    return pl.pallas_call(
        _mm_body,
        grid=(pl.cdiv(V, vt),),
        in_specs=[
            pl.BlockSpec((B, D), lambda i: (0, 0)),
            pl.BlockSpec((vt, D), lambda i: (i, 0)),
            pl.BlockSpec((1, vt), lambda i: (0, i)),
        ],
        out_specs=pl.BlockSpec((B, vt), lambda i: (0, i)),
        out_shape=jax.ShapeDtypeStruct((B, V), jnp.float32),
        compiler_params=pltpu.CompilerParams(
            dimension_semantics=("arbitrary",),
        ),
    )


def kernel(x, embed, W, b):
    B, L = x.shape
    V, D = embed.shape
    s = _build_bag(B, L, V, D)(x.reshape(B * L), embed)
    return _build_mm(B, V, D)(s, W, b.reshape(1, V))


# speedup vs baseline: 1.0093x; 1.0093x over previous
import functools

import jax
import jax.numpy as jnp
from jax import lax
from jax.experimental import pallas as pl
from jax.experimental.pallas import tpu as pltpu
from jax.experimental.pallas import tpu_sc as plsc

_LANES = 16
_PADW = 128


@functools.lru_cache(maxsize=None)
def _build_bag(B, L, V, D):
    info = plsc.get_sparse_core_info()
    NW = info.num_cores * info.num_subcores
    assert B % NW == 0
    b_per_w = B // NW
    rows_chunk = 8
    n_chunks = b_per_w // rows_chunk
    n_col = D // _LANES

    mesh = plsc.VectorSubcoreMesh(core_axis_name="c", subcore_axis_name="s")

    @functools.partial(
        pl.kernel,
        mesh=mesh,
        out_type=jax.ShapeDtypeStruct((B, D), jnp.float32),
        scratch_types=[
            pltpu.VMEM((b_per_w, L), jnp.int32),
            pltpu.VMEM((rows_chunk * L, _PADW), jnp.float32),
            pltpu.VMEM((rows_chunk * L, _PADW), jnp.float32),
            pltpu.VMEM((b_per_w, D), jnp.float32),
            pltpu.SemaphoreType.DMA,
            pltpu.SemaphoreType.DMA,
        ],
    )
    def bag(x_hbm, table_hbm, out_hbm, idx_v, rows0, rows1, acc_v, sem0, sem1):
        wid = lax.axis_index("s") * info.num_cores + lax.axis_index("c")
        base = wid * b_per_w
        pltpu.sync_copy(x_hbm.at[pl.ds(base, b_per_w)], idx_v)

        bufs = (rows0, rows1)
        sems = (sem0, sem1)

        def fire(chunk, buf, sem):
            for r in range(rows_chunk):
                pltpu.async_copy(
                    table_hbm.at[idx_v.at[chunk * rows_chunk + r]],
                    buf.at[pl.ds(r * L, L)],
                    sem,
                )

        def drain(buf, sem):
            for r in range(rows_chunk):
                pltpu.make_async_copy(
                    table_hbm.at[idx_v.at[r]],
                    buf.at[pl.ds(r * L, L)],
                    sem,
                ).wait()

        def accumulate(chunk, buf):
            def body_i(i, _):
                r0 = i * L

                def body_j(j, acc):
                    r = r0 + j
                    return tuple(
                        acc[k] + buf[r, pl.ds(_LANES * k, _LANES)]
                        for k in range(n_col)
                    )

                acc = lax.fori_loop(
                    0, L, body_j,
                    tuple(jnp.zeros((_LANES,), jnp.float32)
                          for _ in range(n_col)),
                )
                for k in range(n_col):
                    acc_v[chunk * rows_chunk + i, pl.ds(_LANES * k, _LANES)] = acc[k]
                return 0

            lax.fori_loop(0, rows_chunk, body_i, 0)

        fire(0, bufs[0], sems[0])
        for c in range(n_chunks):
            if c + 1 < n_chunks:
                fire(c + 1, bufs[(c + 1) % 2], sems[(c + 1) % 2])
            drain(bufs[c % 2], sems[c % 2])
            accumulate(c, bufs[c % 2])
        pltpu.sync_copy(acc_v, out_hbm.at[pl.ds(base, b_per_w)])

    return bag


def _mm_body(s_ref, w_ref, b_ref, o_ref):
    o_ref[...] = lax.dot_general(
        s_ref[...], w_ref[...],
        dimension_numbers=(((1,), (1,)), ((), ())),
        preferred_element_type=jnp.float32,
    ) + b_ref[...]


@functools.lru_cache(maxsize=None)
def _build_mm(B, V, D, vt=2048):
    return pl.pallas_call(
        _mm_body,
        grid=(pl.cdiv(V, vt),),
        in_specs=[
            pl.BlockSpec((B, D), lambda i: (0, 0)),
            pl.BlockSpec((vt, D), lambda i: (i, 0)),
            pl.BlockSpec((1, vt), lambda i: (0, i)),
        ],
        out_specs=pl.BlockSpec((B, vt), lambda i: (0, i)),
        out_shape=jax.ShapeDtypeStruct((B, V), jnp.float32),
        compiler_params=pltpu.CompilerParams(
            dimension_semantics=("arbitrary",),
        ),
    )


def kernel(x, embed, W, b):
    B, L = x.shape
    V, D = embed.shape
    table = jnp.pad(embed, ((0, 0), (0, _PADW - D)))
    s = _build_bag(B, L, V, D)(x, table)
    return _build_mm(B, V, D)(s, W, b.reshape(1, V))


# speedup vs baseline: 2.3197x; 2.2983x over previous
import functools

import jax
import jax.numpy as jnp
from jax import lax
from jax.experimental import pallas as pl
from jax.experimental.pallas import tpu as pltpu
from jax.experimental.pallas import tpu_sc as plsc

_LANES = 16
_PADW = 128


@functools.lru_cache(maxsize=None)
def _build_bag(B, L, V, D):
    info = plsc.get_sparse_core_info()
    NW = info.num_cores * info.num_subcores
    assert B % NW == 0
    b_per_w = B // NW
    rows_chunk = 8
    n_chunks = b_per_w // rows_chunk
    n_col = D // _LANES

    mesh = plsc.VectorSubcoreMesh(core_axis_name="c", subcore_axis_name="s")

    @functools.partial(
        pl.kernel,
        mesh=mesh,
        out_type=jax.ShapeDtypeStruct((B, D), jnp.float32),
        scratch_types=[
            pltpu.VMEM((b_per_w, L), jnp.int32),
            pltpu.VMEM((rows_chunk * L, _PADW), jnp.float32),
            pltpu.VMEM((rows_chunk * L, _PADW), jnp.float32),
            pltpu.VMEM((b_per_w, D), jnp.float32),
            pltpu.SemaphoreType.DMA,
            pltpu.SemaphoreType.DMA,
        ],
    )
    def bag(x_hbm, table_hbm, out_hbm, idx_v, rows0, rows1, acc_v, sem0, sem1):
        wid = lax.axis_index("s") * info.num_cores + lax.axis_index("c")
        base = wid * b_per_w
        pltpu.sync_copy(x_hbm.at[pl.ds(base, b_per_w)], idx_v)

        bufs = (rows0, rows1)
        sems = (sem0, sem1)

        def fire(chunk, buf, sem):
            for r in range(rows_chunk):
                pltpu.async_copy(
                    table_hbm.at[idx_v.at[chunk * rows_chunk + r]],
                    buf.at[pl.ds(r * L, L)],
                    sem,
                )

        def drain(buf, sem):
            for r in range(rows_chunk):
                pltpu.make_async_copy(
                    table_hbm.at[idx_v.at[r]],
                    buf.at[pl.ds(r * L, L)],
                    sem,
                ).wait()

        def accumulate(chunk, buf):
            def body_i(i, _):
                r0 = i * L

                def body_j(j, acc):
                    r = r0 + j
                    return tuple(
                        acc[k] + buf[r, pl.ds(_LANES * k, _LANES)]
                        for k in range(n_col)
                    )

                acc = lax.fori_loop(
                    0, L, body_j,
                    tuple(jnp.zeros((_LANES,), jnp.float32)
                          for _ in range(n_col)),
                )
                for k in range(n_col):
                    acc_v[chunk * rows_chunk + i, pl.ds(_LANES * k, _LANES)] = acc[k]
                return 0

            lax.fori_loop(0, rows_chunk, body_i, 0)

        fire(0, bufs[0], sems[0])
        for c in range(n_chunks):
            if c + 1 < n_chunks:
                fire(c + 1, bufs[(c + 1) % 2], sems[(c + 1) % 2])
            drain(bufs[c % 2], sems[c % 2])
            accumulate(c, bufs[c % 2])
        pltpu.sync_copy(acc_v, out_hbm.at[pl.ds(base, b_per_w)])

    return bag


def _mm_body(wt_ref, s_ref, b_ref, o_ref):
    o_ref[...] = lax.dot_general(
        wt_ref[...], s_ref[...],
        dimension_numbers=(((0,), (1,)), ((), ())),
        preferred_element_type=jnp.float32,
    ) + b_ref[...]


@functools.lru_cache(maxsize=None)
def _build_mm(B, V, D, vt=2048):
    return pl.pallas_call(
        _mm_body,
        grid=(pl.cdiv(V, vt),),
        in_specs=[
            pl.BlockSpec((D, vt), lambda i: (0, i)),
            pl.BlockSpec((B, D), lambda i: (0, 0)),
            pl.BlockSpec((vt, 1), lambda i: (i, 0)),
        ],
        out_specs=pl.BlockSpec((vt, B), lambda i: (i, 0)),
        out_shape=jax.ShapeDtypeStruct((V, B), jnp.float32),
        compiler_params=pltpu.CompilerParams(
            dimension_semantics=("arbitrary",),
        ),
    )


def kernel(x, embed, W, b):
    B, L = x.shape
    V, D = embed.shape
    table = jnp.pad(embed, ((0, 0), (0, _PADW - D)))
    s = _build_bag(B, L, V, D)(x, table)
    out_t = _build_mm(B, V, D)(W.T, s, b.reshape(V, 1))
    return out_t.T


# speedup vs baseline: 2.7690x; 1.1937x over previous
import functools

import jax
import jax.numpy as jnp
from jax import lax
from jax.experimental import pallas as pl
from jax.experimental.pallas import tpu as pltpu
from jax.experimental.pallas import tpu_sc as plsc

_LANES = 16
_PADW = 128


@functools.lru_cache(maxsize=None)
def _build_bag(B, L, V, D):
    info = plsc.get_sparse_core_info()
    NW = info.num_cores * info.num_subcores
    assert B % NW == 0
    b_per_w = B // NW
    rows_chunk = 8
    n_chunks = b_per_w // rows_chunk
    n_col = D // _LANES

    mesh = plsc.VectorSubcoreMesh(core_axis_name="c", subcore_axis_name="s")

    @functools.partial(
        pl.kernel,
        mesh=mesh,
        out_type=jax.ShapeDtypeStruct((B, D), jnp.float32),
        scratch_types=[
            pltpu.VMEM((b_per_w, L), jnp.int32),
            pltpu.VMEM((rows_chunk * L, _PADW), jnp.float32),
            pltpu.VMEM((rows_chunk * L, _PADW), jnp.float32),
            pltpu.VMEM((b_per_w, D), jnp.float32),
            pltpu.SemaphoreType.DMA,
            pltpu.SemaphoreType.DMA,
        ],
    )
    def bag(x_hbm, table_hbm, out_hbm, idx_v, rows0, rows1, acc_v, sem0, sem1):
        wid = lax.axis_index("s") * info.num_cores + lax.axis_index("c")
        base = wid * b_per_w
        pltpu.sync_copy(x_hbm.at[pl.ds(base, b_per_w)], idx_v)

        bufs = (rows0, rows1)
        sems = (sem0, sem1)

        def fire(chunk, buf, sem):
            for r in range(rows_chunk):
                pltpu.async_copy(
                    table_hbm.at[idx_v.at[chunk * rows_chunk + r]],
                    buf.at[pl.ds(r * L, L)],
                    sem,
                )

        def drain(buf, sem):
            for r in range(rows_chunk):
                pltpu.make_async_copy(
                    table_hbm.at[idx_v.at[r]],
                    buf.at[pl.ds(r * L, L)],
                    sem,
                ).wait()

        def accumulate(chunk, buf):
            def body_i(i, _):
                r0 = i * L

                def body_j(j, acc):
                    r = r0 + j
                    return tuple(
                        acc[k] + buf[r, pl.ds(_LANES * k, _LANES)]
                        for k in range(n_col)
                    )

                acc = lax.fori_loop(
                    0, L, body_j,
                    tuple(jnp.zeros((_LANES,), jnp.float32)
                          for _ in range(n_col)),
                )
                for k in range(n_col):
                    acc_v[chunk * rows_chunk + i, pl.ds(_LANES * k, _LANES)] = acc[k]
                return 0

            lax.fori_loop(0, rows_chunk, body_i, 0)

        fire(0, bufs[0], sems[0])
        for c in range(n_chunks):
            if c + 1 < n_chunks:
                fire(c + 1, bufs[(c + 1) % 2], sems[(c + 1) % 2])
            drain(bufs[c % 2], sems[c % 2])
            accumulate(c, bufs[c % 2])
        pltpu.sync_copy(acc_v, out_hbm.at[pl.ds(base, b_per_w)])

    return bag


def _mm_body(wt_ref, s_ref, b_ref, o_ref):
    o_ref[...] = lax.dot_general(
        wt_ref[...], s_ref[...],
        dimension_numbers=(((0,), (1,)), ((), ())),
        preferred_element_type=jnp.float32,
    ) + jnp.swapaxes(b_ref[...], 0, 1)


@functools.lru_cache(maxsize=None)
def _build_mm(B, V, D, vt=2048):
    return pl.pallas_call(
        _mm_body,
        grid=(pl.cdiv(V, vt),),
        in_specs=[
            pl.BlockSpec((D, vt), lambda i: (0, i)),
            pl.BlockSpec((B, D), lambda i: (0, 0)),
            pl.BlockSpec((1, vt), lambda i: (0, i)),
        ],
        out_specs=pl.BlockSpec((vt, B), lambda i: (i, 0)),
        out_shape=jax.ShapeDtypeStruct((V, B), jnp.float32),
        compiler_params=pltpu.CompilerParams(
            dimension_semantics=("arbitrary",),
        ),
    )


def kernel(x, embed, W, b):
    B, L = x.shape
    V, D = embed.shape
    table = jnp.pad(embed, ((0, 0), (0, _PADW - D)))
    s = _build_bag(B, L, V, D)(x, table)
    out_t = _build_mm(B, V, D)(W.T, s, b.reshape(1, V))
    return out_t.T


# speedup vs baseline: 2.8168x; 1.0173x over previous
import functools

import jax
import jax.numpy as jnp
from jax import lax
from jax.experimental import pallas as pl
from jax.experimental.pallas import tpu as pltpu
from jax.experimental.pallas import tpu_sc as plsc

_LANES = 16
_PADW = 128


def _prep_body(et_ref, o_ref):
    D = et_ref.shape[0]
    row = lax.broadcasted_iota(jnp.int32, (D, D), 0)
    col = lax.broadcasted_iota(jnp.int32, (D, D), 1)
    eye = jnp.where(row == col, 1.0, 0.0).astype(jnp.float32)
    blk_t = lax.dot_general(
        et_ref[...], eye,
        dimension_numbers=(((0,), (0,)), ((), ())),
        preferred_element_type=jnp.float32,
    )
    o_ref[:, pl.ds(0, D)] = blk_t
    o_ref[:, pl.ds(D, _PADW - D)] = jnp.zeros(
        (o_ref.shape[0], _PADW - D), jnp.float32)


@functools.lru_cache(maxsize=None)
def _build_prep(V, D, vt=2048):
    return pl.pallas_call(
        _prep_body,
        grid=(pl.cdiv(V, vt),),
        in_specs=[pl.BlockSpec((D, vt), lambda i: (0, i))],
        out_specs=pl.BlockSpec((vt, _PADW), lambda i: (i, 0)),
        out_shape=jax.ShapeDtypeStruct((V, _PADW), jnp.float32),
        compiler_params=pltpu.CompilerParams(
            dimension_semantics=("arbitrary",),
        ),
    )


@functools.lru_cache(maxsize=None)
def _build_bag(B, L, V, D):
    info = plsc.get_sparse_core_info()
    NW = info.num_cores * info.num_subcores
    assert B % NW == 0
    b_per_w = B // NW
    rows_chunk = 8
    n_chunks = b_per_w // rows_chunk
    n_col = D // _LANES

    mesh = plsc.VectorSubcoreMesh(core_axis_name="c", subcore_axis_name="s")

    @functools.partial(
        pl.kernel,
        mesh=mesh,
        out_type=jax.ShapeDtypeStruct((B, D), jnp.float32),
        scratch_types=[
            pltpu.VMEM((b_per_w, L), jnp.int32),
            pltpu.VMEM((rows_chunk * L, _PADW), jnp.float32),
            pltpu.VMEM((rows_chunk * L, _PADW), jnp.float32),
            pltpu.VMEM((b_per_w, D), jnp.float32),
            pltpu.SemaphoreType.DMA,
            pltpu.SemaphoreType.DMA,
        ],
    )
    def bag(x_hbm, table_hbm, out_hbm, idx_v, rows0, rows1, acc_v, sem0, sem1):
        wid = lax.axis_index("s") * info.num_cores + lax.axis_index("c")
        base = wid * b_per_w
        pltpu.sync_copy(x_hbm.at[pl.ds(base, b_per_w)], idx_v)

        bufs = (rows0, rows1)
        sems = (sem0, sem1)

        def fire(chunk, buf, sem):
            for r in range(rows_chunk):
                pltpu.async_copy(
                    table_hbm.at[idx_v.at[chunk * rows_chunk + r]],
                    buf.at[pl.ds(r * L, L)],
                    sem,
                )

        def drain(buf, sem):
            for r in range(rows_chunk):
                pltpu.make_async_copy(
                    table_hbm.at[idx_v.at[r]],
                    buf.at[pl.ds(r * L, L)],
                    sem,
                ).wait()

        def accumulate(chunk, buf):
            def body_i(i, _):
                r0 = i * L

                def body_j(j, acc):
                    r = r0 + j
                    return tuple(
                        acc[k] + buf[r, pl.ds(_LANES * k, _LANES)]
                        for k in range(n_col)
                    )

                acc = lax.fori_loop(
                    0, L, body_j,
                    tuple(jnp.zeros((_LANES,), jnp.float32)
                          for _ in range(n_col)),
                )
                for k in range(n_col):
                    acc_v[chunk * rows_chunk + i, pl.ds(_LANES * k, _LANES)] = acc[k]
                return 0

            lax.fori_loop(0, rows_chunk, body_i, 0)

        fire(0, bufs[0], sems[0])
        for c in range(n_chunks):
            if c + 1 < n_chunks:
                fire(c + 1, bufs[(c + 1) % 2], sems[(c + 1) % 2])
            drain(bufs[c % 2], sems[c % 2])
            accumulate(c, bufs[c % 2])
        pltpu.sync_copy(acc_v, out_hbm.at[pl.ds(base, b_per_w)])

    return bag


def _mm_body(wt_ref, s_ref, b_ref, o_ref):
    o_ref[...] = lax.dot_general(
        wt_ref[...], s_ref[...],
        dimension_numbers=(((0,), (1,)), ((), ())),
        preferred_element_type=jnp.float32,
    ) + jnp.swapaxes(b_ref[...], 0, 1)


@functools.lru_cache(maxsize=None)
def _build_mm(B, V, D, vt=2048):
    return pl.pallas_call(
        _mm_body,
        grid=(pl.cdiv(V, vt),),
        in_specs=[
            pl.BlockSpec((D, vt), lambda i: (0, i)),
            pl.BlockSpec((B, D), lambda i: (0, 0)),
            pl.BlockSpec((1, vt), lambda i: (0, i)),
        ],
        out_specs=pl.BlockSpec((vt, B), lambda i: (i, 0)),
        out_shape=jax.ShapeDtypeStruct((V, B), jnp.float32),
        compiler_params=pltpu.CompilerParams(
            dimension_semantics=("arbitrary",),
        ),
    )


def kernel(x, embed, W, b):
    B, L = x.shape
    V, D = embed.shape
    table = _build_prep(V, D)(embed.T)
    s = _build_bag(B, L, V, D)(x, table)
    out_t = _build_mm(B, V, D)(W.T, s, b.reshape(1, V))
    return out_t.T


# speedup vs baseline: 2.8365x; 1.0070x over previous
import functools

import jax
import jax.numpy as jnp
from jax import lax
from jax.experimental import pallas as pl
from jax.experimental.pallas import tpu as pltpu
from jax.experimental.pallas import tpu_sc as plsc

_LANES = 16
_PADW = 128


def _prep_body(et_ref, o_ref):
    D = et_ref.shape[0]
    blk_t = jnp.swapaxes(et_ref[...], 0, 1)
    o_ref[:, pl.ds(0, D)] = blk_t
    o_ref[:, pl.ds(D, _PADW - D)] = jnp.zeros(
        (o_ref.shape[0], _PADW - D), jnp.float32)


@functools.lru_cache(maxsize=None)
def _build_prep(V, D, vt=2048):
    return pl.pallas_call(
        _prep_body,
        grid=(pl.cdiv(V, vt),),
        in_specs=[pl.BlockSpec((D, vt), lambda i: (0, i))],
        out_specs=pl.BlockSpec((vt, _PADW), lambda i: (i, 0)),
        out_shape=jax.ShapeDtypeStruct((V, _PADW), jnp.float32),
        compiler_params=pltpu.CompilerParams(
            dimension_semantics=("arbitrary",),
        ),
    )


@functools.lru_cache(maxsize=None)
def _build_bag(B, L, V, D):
    info = plsc.get_sparse_core_info()
    NW = info.num_cores * info.num_subcores
    assert B % NW == 0
    b_per_w = B // NW
    rows_chunk = 8
    n_chunks = b_per_w // rows_chunk
    n_col = D // _LANES

    mesh = plsc.VectorSubcoreMesh(core_axis_name="c", subcore_axis_name="s")

    @functools.partial(
        pl.kernel,
        mesh=mesh,
        out_type=jax.ShapeDtypeStruct((B, D), jnp.float32),
        scratch_types=[
            pltpu.VMEM((b_per_w, L), jnp.int32),
            pltpu.VMEM((rows_chunk * L, _PADW), jnp.float32),
            pltpu.VMEM((rows_chunk * L, _PADW), jnp.float32),
            pltpu.VMEM((b_per_w, D), jnp.float32),
            pltpu.SemaphoreType.DMA,
            pltpu.SemaphoreType.DMA,
        ],
    )
    def bag(x_hbm, table_hbm, out_hbm, idx_v, rows0, rows1, acc_v, sem0, sem1):
        wid = lax.axis_index("s") * info.num_cores + lax.axis_index("c")
        base = wid * b_per_w
        pltpu.sync_copy(x_hbm.at[pl.ds(base, b_per_w)], idx_v)

        bufs = (rows0, rows1)
        sems = (sem0, sem1)

        def fire(chunk, buf, sem):
            for r in range(rows_chunk):
                pltpu.async_copy(
                    table_hbm.at[idx_v.at[chunk * rows_chunk + r]],
                    buf.at[pl.ds(r * L, L)],
                    sem,
                )

        def drain(buf, sem):
            for r in range(rows_chunk):
                pltpu.make_async_copy(
                    table_hbm.at[idx_v.at[r]],
                    buf.at[pl.ds(r * L, L)],
                    sem,
                ).wait()

        def accumulate(chunk, buf):
            def body_i(i, _):
                r0 = i * L

                def body_j(j, acc):
                    r = r0 + j
                    return tuple(
                        acc[k] + buf[r, pl.ds(_LANES * k, _LANES)]
                        for k in range(n_col)
                    )

                acc = lax.fori_loop(
                    0, L, body_j,
                    tuple(jnp.zeros((_LANES,), jnp.float32)
                          for _ in range(n_col)),
                )
                for k in range(n_col):
                    acc_v[chunk * rows_chunk + i, pl.ds(_LANES * k, _LANES)] = acc[k]
                return 0

            lax.fori_loop(0, rows_chunk, body_i, 0)

        fire(0, bufs[0], sems[0])
        for c in range(n_chunks):
            if c + 1 < n_chunks:
                fire(c + 1, bufs[(c + 1) % 2], sems[(c + 1) % 2])
            drain(bufs[c % 2], sems[c % 2])
            accumulate(c, bufs[c % 2])
        pltpu.sync_copy(acc_v, out_hbm.at[pl.ds(base, b_per_w)])

    return bag


def _mm_body(wt_ref, s_ref, b_ref, o_ref):
    o_ref[...] = lax.dot_general(
        wt_ref[...], s_ref[...],
        dimension_numbers=(((0,), (1,)), ((), ())),
        preferred_element_type=jnp.float32,
    ) + jnp.swapaxes(b_ref[...], 0, 1)


@functools.lru_cache(maxsize=None)
def _build_mm(B, V, D, vt=2048):
    return pl.pallas_call(
        _mm_body,
        grid=(pl.cdiv(V, vt),),
        in_specs=[
            pl.BlockSpec((D, vt), lambda i: (0, i)),
            pl.BlockSpec((B, D), lambda i: (0, 0)),
            pl.BlockSpec((1, vt), lambda i: (0, i)),
        ],
        out_specs=pl.BlockSpec((vt, B), lambda i: (i, 0)),
        out_shape=jax.ShapeDtypeStruct((V, B), jnp.float32),
        compiler_params=pltpu.CompilerParams(
            dimension_semantics=("arbitrary",),
        ),
    )


def kernel(x, embed, W, b):
    B, L = x.shape
    V, D = embed.shape
    table = _build_prep(V, D)(embed.T)
    s = _build_bag(B, L, V, D)(x, table)
    out_t = _build_mm(B, V, D)(W.T, s, b.reshape(1, V))
    return out_t.T


# speedup vs baseline: 2.8746x; 1.0134x over previous
import functools

import jax
import jax.numpy as jnp
from jax import lax
from jax.experimental import pallas as pl
from jax.experimental.pallas import tpu as pltpu
from jax.experimental.pallas import tpu_sc as plsc

_LANES = 16
_PADW = 128


def _prep_body(et_ref, o_ref):
    D = et_ref.shape[0]
    o_ref[:, pl.ds(0, D)] = jnp.swapaxes(et_ref[...], 0, 1)


@functools.lru_cache(maxsize=None)
def _build_prep(V, D, vt=2048):
    return pl.pallas_call(
        _prep_body,
        grid=(pl.cdiv(V, vt),),
        in_specs=[pl.BlockSpec((D, vt), lambda i: (0, i))],
        out_specs=pl.BlockSpec((vt, _PADW), lambda i: (i, 0)),
        out_shape=jax.ShapeDtypeStruct((V, _PADW), jnp.float32),
        compiler_params=pltpu.CompilerParams(
            dimension_semantics=("arbitrary",),
        ),
    )


@functools.lru_cache(maxsize=None)
def _build_bag(B, L, V, D):
    info = plsc.get_sparse_core_info()
    NW = info.num_cores * info.num_subcores
    assert B % NW == 0
    b_per_w = B // NW
    rows_chunk = 8
    n_chunks = b_per_w // rows_chunk
    n_col = D // _LANES

    mesh = plsc.VectorSubcoreMesh(core_axis_name="c", subcore_axis_name="s")

    @functools.partial(
        pl.kernel,
        mesh=mesh,
        out_type=jax.ShapeDtypeStruct((B, D), jnp.float32),
        scratch_types=[
            pltpu.VMEM((b_per_w, L), jnp.int32),
            pltpu.VMEM((rows_chunk * L, _PADW), jnp.float32),
            pltpu.VMEM((rows_chunk * L, _PADW), jnp.float32),
            pltpu.VMEM((b_per_w, D), jnp.float32),
            pltpu.SemaphoreType.DMA,
            pltpu.SemaphoreType.DMA,
        ],
    )
    def bag(x_hbm, table_hbm, out_hbm, idx_v, rows0, rows1, acc_v, sem0, sem1):
        wid = lax.axis_index("s") * info.num_cores + lax.axis_index("c")
        base = wid * b_per_w
        pltpu.sync_copy(x_hbm.at[pl.ds(base, b_per_w)], idx_v)

        bufs = (rows0, rows1)
        sems = (sem0, sem1)

        def fire(chunk, buf, sem):
            for r in range(rows_chunk):
                pltpu.async_copy(
                    table_hbm.at[idx_v.at[chunk * rows_chunk + r]],
                    buf.at[pl.ds(r * L, L)],
                    sem,
                )

        def drain(buf, sem):
            for r in range(rows_chunk):
                pltpu.make_async_copy(
                    table_hbm.at[idx_v.at[r]],
                    buf.at[pl.ds(r * L, L)],
                    sem,
                ).wait()

        def accumulate(chunk, buf):
            def body_i(i, _):
                r0 = i * L

                def body_j(j, acc):
                    r = r0 + j
                    return tuple(
                        acc[k] + buf[r, pl.ds(_LANES * k, _LANES)]
                        for k in range(n_col)
                    )

                acc = lax.fori_loop(
                    0, L, body_j,
                    tuple(jnp.zeros((_LANES,), jnp.float32)
                          for _ in range(n_col)),
                )
                for k in range(n_col):
                    acc_v[chunk * rows_chunk + i, pl.ds(_LANES * k, _LANES)] = acc[k]
                return 0

            lax.fori_loop(0, rows_chunk, body_i, 0)

        fire(0, bufs[0], sems[0])
        for c in range(n_chunks):
            if c + 1 < n_chunks:
                fire(c + 1, bufs[(c + 1) % 2], sems[(c + 1) % 2])
            drain(bufs[c % 2], sems[c % 2])
            accumulate(c, bufs[c % 2])
        pltpu.sync_copy(acc_v, out_hbm.at[pl.ds(base, b_per_w)])

    return bag


def _mm_body(wt_ref, s_ref, b_ref, o_ref):
    o_ref[...] = lax.dot_general(
        wt_ref[...], s_ref[...],
        dimension_numbers=(((0,), (1,)), ((), ())),
        preferred_element_type=jnp.float32,
    ) + jnp.swapaxes(b_ref[...], 0, 1)


@functools.lru_cache(maxsize=None)
def _build_mm(B, V, D, vt=4096):
    return pl.pallas_call(
        _mm_body,
        grid=(pl.cdiv(V, vt),),
        in_specs=[
            pl.BlockSpec((D, vt), lambda i: (0, i)),
            pl.BlockSpec((B, D), lambda i: (0, 0)),
            pl.BlockSpec((1, vt), lambda i: (0, i)),
        ],
        out_specs=pl.BlockSpec((vt, B), lambda i: (i, 0)),
        out_shape=jax.ShapeDtypeStruct((V, B), jnp.float32),
        compiler_params=pltpu.CompilerParams(
            dimension_semantics=("arbitrary",),
        ),
    )


def kernel(x, embed, W, b):
    B, L = x.shape
    V, D = embed.shape
    table = _build_prep(V, D)(embed.T)
    s = _build_bag(B, L, V, D)(x, table)
    out_t = _build_mm(B, V, D)(W.T, s, b.reshape(1, V))
    return out_t.T


# speedup vs baseline: 3.1723x; 1.1036x over previous
import functools

import jax
import jax.numpy as jnp
from jax import lax
from jax.experimental import pallas as pl
from jax.experimental.pallas import tpu as pltpu
from jax.experimental.pallas import tpu_sc as plsc

_LANES = 16
_PADW = 128


def _prep_body(et_ref, o_ref):
    D = et_ref.shape[0]
    o_ref[:, pl.ds(0, D)] = jnp.swapaxes(et_ref[...], 0, 1)


@functools.lru_cache(maxsize=None)
def _build_prep(V, D, vt=8192):
    return pl.pallas_call(
        _prep_body,
        grid=(pl.cdiv(V, vt),),
        in_specs=[pl.BlockSpec((D, vt), lambda i: (0, i))],
        out_specs=pl.BlockSpec((vt, _PADW), lambda i: (i, 0)),
        out_shape=jax.ShapeDtypeStruct((V, _PADW), jnp.float32),
        compiler_params=pltpu.CompilerParams(
            dimension_semantics=("arbitrary",),
        ),
    )


@functools.lru_cache(maxsize=None)
def _build_bag(B, L, V, D):
    info = plsc.get_sparse_core_info()
    NW = info.num_cores * info.num_subcores
    assert B % NW == 0
    b_per_w = B // NW
    rows_chunk = 8
    n_chunks = b_per_w // rows_chunk
    n_col = D // _LANES

    mesh = plsc.VectorSubcoreMesh(core_axis_name="c", subcore_axis_name="s")

    @functools.partial(
        pl.kernel,
        mesh=mesh,
        out_type=jax.ShapeDtypeStruct((B, D), jnp.float32),
        scratch_types=[
            pltpu.VMEM((b_per_w, L), jnp.int32),
            pltpu.VMEM((rows_chunk * L, _PADW), jnp.float32),
            pltpu.VMEM((rows_chunk * L, _PADW), jnp.float32),
            pltpu.VMEM((b_per_w, D), jnp.float32),
            pltpu.SemaphoreType.DMA,
            pltpu.SemaphoreType.DMA,
        ],
    )
    def bag(x_hbm, table_hbm, out_hbm, idx_v, rows0, rows1, acc_v, sem0, sem1):
        wid = lax.axis_index("s") * info.num_cores + lax.axis_index("c")
        base = wid * b_per_w
        pltpu.sync_copy(x_hbm.at[pl.ds(base, b_per_w)], idx_v)

        bufs = (rows0, rows1)
        sems = (sem0, sem1)

        def fire(chunk, buf, sem):
            for r in range(rows_chunk):
                pltpu.async_copy(
                    table_hbm.at[idx_v.at[chunk * rows_chunk + r]],
                    buf.at[pl.ds(r * L, L)],
                    sem,
                )

        def drain(buf, sem):
            for r in range(rows_chunk):
                pltpu.make_async_copy(
                    table_hbm.at[idx_v.at[r]],
                    buf.at[pl.ds(r * L, L)],
                    sem,
                ).wait()

        def accumulate(chunk, buf):
            def body_i(i, _):
                r0 = i * L

                def body_j(j, acc):
                    r = r0 + j
                    return tuple(
                        acc[k] + buf[r, pl.ds(_LANES * k, _LANES)]
                        for k in range(n_col)
                    )

                acc = lax.fori_loop(
                    0, L, body_j,
                    tuple(jnp.zeros((_LANES,), jnp.float32)
                          for _ in range(n_col)),
                    unroll=5,
                )
                for k in range(n_col):
                    acc_v[chunk * rows_chunk + i, pl.ds(_LANES * k, _LANES)] = acc[k]
                return 0

            lax.fori_loop(0, rows_chunk, body_i, 0)

        fire(0, bufs[0], sems[0])
        for c in range(n_chunks):
            if c + 1 < n_chunks:
                fire(c + 1, bufs[(c + 1) % 2], sems[(c + 1) % 2])
            drain(bufs[c % 2], sems[c % 2])
            accumulate(c, bufs[c % 2])
        pltpu.sync_copy(acc_v, out_hbm.at[pl.ds(base, b_per_w)])

    return bag


def _mm_body(wt_ref, s_ref, b_ref, o_ref):
    o_ref[...] = lax.dot_general(
        wt_ref[...], s_ref[...],
        dimension_numbers=(((0,), (1,)), ((), ())),
        preferred_element_type=jnp.float32,
    ) + jnp.swapaxes(b_ref[...], 0, 1)


@functools.lru_cache(maxsize=None)
def _build_mm(B, V, D, vt=4096):
    return pl.pallas_call(
        _mm_body,
        grid=(pl.cdiv(V, vt),),
        in_specs=[
            pl.BlockSpec((D, vt), lambda i: (0, i)),
            pl.BlockSpec((B, D), lambda i: (0, 0)),
            pl.BlockSpec((1, vt), lambda i: (0, i)),
        ],
        out_specs=pl.BlockSpec((vt, B), lambda i: (i, 0)),
        out_shape=jax.ShapeDtypeStruct((V, B), jnp.float32),
        compiler_params=pltpu.CompilerParams(
            dimension_semantics=("arbitrary",),
        ),
    )


def kernel(x, embed, W, b):
    B, L = x.shape
    V, D = embed.shape
    table = _build_prep(V, D)(embed.T)
    s = _build_bag(B, L, V, D)(x, table)
    out_t = _build_mm(B, V, D)(W.T, s, b.reshape(1, V))
    return out_t.T


# speedup vs baseline: 3.1898x; 1.0055x over previous
import functools

import jax
import jax.numpy as jnp
from jax import lax
from jax.experimental import pallas as pl
from jax.experimental.pallas import tpu as pltpu
from jax.experimental.pallas import tpu_sc as plsc

_LANES = 16
_PADW = 128


def _prep_body(et_ref, o_ref):
    D = et_ref.shape[0]
    o_ref[:, pl.ds(0, D)] = jnp.swapaxes(et_ref[...], 0, 1)


@functools.lru_cache(maxsize=None)
def _build_prep(V, D, vt=16384):
    return pl.pallas_call(
        _prep_body,
        grid=(pl.cdiv(V, vt),),
        in_specs=[pl.BlockSpec((D, vt), lambda i: (0, i))],
        out_specs=pl.BlockSpec((vt, _PADW), lambda i: (i, 0)),
        out_shape=jax.ShapeDtypeStruct((V, _PADW), jnp.float32),
        compiler_params=pltpu.CompilerParams(
            dimension_semantics=("arbitrary",),
        ),
    )


@functools.lru_cache(maxsize=None)
def _build_bag(B, L, V, D):
    info = plsc.get_sparse_core_info()
    NW = info.num_cores * info.num_subcores
    assert B % NW == 0
    b_per_w = B // NW
    rows_chunk = 8
    n_chunks = b_per_w // rows_chunk
    n_col = D // _LANES

    mesh = plsc.VectorSubcoreMesh(core_axis_name="c", subcore_axis_name="s")

    @functools.partial(
        pl.kernel,
        mesh=mesh,
        out_type=jax.ShapeDtypeStruct((B, D), jnp.float32),
        scratch_types=[
            pltpu.VMEM((b_per_w, L), jnp.int32),
            pltpu.VMEM((rows_chunk * L, _PADW), jnp.float32),
            pltpu.VMEM((rows_chunk * L, _PADW), jnp.float32),
            pltpu.VMEM((b_per_w, D), jnp.float32),
            pltpu.SemaphoreType.DMA,
            pltpu.SemaphoreType.DMA,
        ],
    )
    def bag(x_hbm, table_hbm, out_hbm, idx_v, rows0, rows1, acc_v, sem0, sem1):
        wid = lax.axis_index("s") * info.num_cores + lax.axis_index("c")
        base = wid * b_per_w
        pltpu.sync_copy(x_hbm.at[pl.ds(base, b_per_w)], idx_v)

        bufs = (rows0, rows1)
        sems = (sem0, sem1)

        def fire(chunk, buf, sem):
            for r in range(rows_chunk):
                pltpu.async_copy(
                    table_hbm.at[idx_v.at[chunk * rows_chunk + r]],
                    buf.at[pl.ds(r * L, L)],
                    sem,
                )

        def drain(buf, sem):
            for r in range(rows_chunk):
                pltpu.make_async_copy(
                    table_hbm.at[idx_v.at[r]],
                    buf.at[pl.ds(r * L, L)],
                    sem,
                ).wait()

        def accumulate(chunk, buf):
            def body_i(i, _):
                r0 = i * L

                def body_j(j, acc):
                    r = r0 + j
                    return tuple(
                        acc[k] + buf[r, pl.ds(_LANES * k, _LANES)]
                        for k in range(n_col)
                    )

                acc = lax.fori_loop(
                    0, L, body_j,
                    tuple(jnp.zeros((_LANES,), jnp.float32)
                          for _ in range(n_col)),
                    unroll=10,
                )
                for k in range(n_col):
                    acc_v[chunk * rows_chunk + i, pl.ds(_LANES * k, _LANES)] = acc[k]
                return 0

            lax.fori_loop(0, rows_chunk, body_i, 0)

        fire(0, bufs[0], sems[0])
        for c in range(n_chunks):
            if c + 1 < n_chunks:
                fire(c + 1, bufs[(c + 1) % 2], sems[(c + 1) % 2])
            drain(bufs[c % 2], sems[c % 2])
            accumulate(c, bufs[c % 2])
        pltpu.sync_copy(acc_v, out_hbm.at[pl.ds(base, b_per_w)])

    return bag


def _mm_body(wt_ref, s_ref, b_ref, o_ref):
    o_ref[...] = lax.dot_general(
        wt_ref[...], s_ref[...],
        dimension_numbers=(((0,), (1,)), ((), ())),
        preferred_element_type=jnp.float32,
    ) + jnp.swapaxes(b_ref[...], 0, 1)


@functools.lru_cache(maxsize=None)
def _build_mm(B, V, D, vt=4096):
    return pl.pallas_call(
        _mm_body,
        grid=(pl.cdiv(V, vt),),
        in_specs=[
            pl.BlockSpec((D, vt), lambda i: (0, i)),
            pl.BlockSpec((B, D), lambda i: (0, 0)),
            pl.BlockSpec((1, vt), lambda i: (0, i)),
        ],
        out_specs=pl.BlockSpec((vt, B), lambda i: (i, 0)),
        out_shape=jax.ShapeDtypeStruct((V, B), jnp.float32),
        compiler_params=pltpu.CompilerParams(
            dimension_semantics=("arbitrary",),
        ),
    )


def kernel(x, embed, W, b):
    B, L = x.shape
    V, D = embed.shape
    table = _build_prep(V, D)(embed.T)
    s = _build_bag(B, L, V, D)(x, table)
    out_t = _build_mm(B, V, D)(W.T, s, b.reshape(1, V))
    return out_t.T
